# baseline (speedup 1.0000x reference)
import jax
import jax.numpy as jnp
from jax import lax
from jax.experimental import pallas as pl
from jax.experimental.pallas import tpu as pltpu

D = 128
H = 64
BLOCK = 16384


def _body(idxa_ref, idxb_ref, ind_ref, wm_ref, wa_ref, bm_ref, ba_ref,
          x_ref, out_ref, k_ref, bmf_ref, baf_ref):
    f32 = jnp.float32

    @pl.when(pl.program_id(0) == 0)
    def _prep():
        idxa = idxa_ref[...].reshape(1, H)
        idxb = idxb_ref[...].reshape(1, H)
        ind = ind_ref[...].reshape(1, H)
        bm = bm_ref[...].reshape(1, H)
        ba = ba_ref[...].reshape(1, H)
        iota_dh = lax.broadcasted_iota(jnp.int32, (D, H), 0)
        ga = (iota_dh == idxa).astype(f32)
        gb = (iota_dh == idxb).astype(f32)
        s = (iota_dh == ind).astype(f32).T
        t_dims = (((1,), (1,)), ((), ()))
        wm_full = lax.dot_general(gb, wm_ref[...], t_dims, preferred_element_type=f32)
        wa_full = lax.dot_general(gb, wa_ref[...], t_dims, preferred_element_type=f32)
        keep = 1.0 - jnp.sum(s, axis=0, keepdims=True)
        iota_r = lax.broadcasted_iota(jnp.int32, (D, D), 0)
        iota_c = lax.broadcasted_iota(jnp.int32, (D, D), 1)
        Wm_f = jnp.dot(wm_full, s, preferred_element_type=f32)
        Ga_f = jnp.dot(ga, s, preferred_element_type=f32)
        M = jnp.where(iota_r == iota_c, keep, 0.0) \
            + jnp.dot(wa_full, s, preferred_element_type=f32)
        k_ref[...] = jnp.concatenate([Wm_f, Ga_f, M], axis=1)
        bmf_ref[...] = jnp.dot(bm, s, preferred_element_type=f32)
        baf_ref[...] = jnp.dot(ba, s, preferred_element_type=f32)

    x = x_ref[...]
    acc = jnp.dot(x, k_ref[...], preferred_element_type=f32)
    out_ref[...] = (acc[:, :D] + bmf_ref[...]) * acc[:, D:2 * D] \
        + acc[:, 2 * D:] + baf_ref[...]


def kernel(x, W_mul, b_mul, W_add, b_add, indices, idx_A, idx_B):
    n = x.shape[0]
    grid = n // BLOCK

    vec = pl.BlockSpec((H,), lambda i: (0,))
    mat = pl.BlockSpec((H, H), lambda i: (0, 0))
    return pl.pallas_call(
        _body,
        grid=(grid,),
        in_specs=[
            vec,
            vec,
            vec,
            mat,
            mat,
            vec,
            vec,
            pl.BlockSpec((BLOCK, D), lambda i: (i, 0)),
        ],
        out_specs=pl.BlockSpec((BLOCK, D), lambda i: (i, 0)),
        out_shape=jax.ShapeDtypeStruct((n, D), jnp.float32),
        scratch_shapes=[
            pltpu.VMEM((D, 3 * D), jnp.float32),
            pltpu.VMEM((1, D), jnp.float32),
            pltpu.VMEM((1, D), jnp.float32),
        ],
    )(idx_A, idx_B, indices, W_mul, W_add, b_mul, b_add, x)


# speedup vs baseline: 1.2339x; 1.2339x over previous
import jax
import jax.numpy as jnp
from jax import lax
from jax.experimental import pallas as pl
from jax.experimental.pallas import tpu as pltpu

D = 128
H = 64
BLOCK = 8192


def _body(idxa_ref, idxb_ref, ind_ref, wm_ref, wa_ref, bm_ref, ba_ref,
          x_ref, out_ref, k_ref, bmf_ref, baf_ref):
    f32 = jnp.float32

    @pl.when(pl.program_id(0) == 0)
    def _prep():
        idxa = idxa_ref[...].reshape(1, H)
        idxb = idxb_ref[...].reshape(1, H)
        ind = ind_ref[...].reshape(1, H)
        bm = bm_ref[...].reshape(1, H)
        ba = ba_ref[...].reshape(1, H)
        iota_dh = lax.broadcasted_iota(jnp.int32, (D, H), 0)
        ga = (iota_dh == idxa).astype(f32)
        gb = (iota_dh == idxb).astype(f32)
        s = (iota_dh == ind).astype(f32).T
        t_dims = (((1,), (1,)), ((), ()))
        wm_full = lax.dot_general(gb, wm_ref[...], t_dims, preferred_element_type=f32)
        wa_full = lax.dot_general(gb, wa_ref[...], t_dims, preferred_element_type=f32)
        keep = 1.0 - jnp.sum(s, axis=0, keepdims=True)
        iota_r = lax.broadcasted_iota(jnp.int32, (D, D), 0)
        iota_c = lax.broadcasted_iota(jnp.int32, (D, D), 1)
        Wm_f = jnp.dot(wm_full, s, preferred_element_type=f32)
        Ga_f = jnp.dot(ga, s, preferred_element_type=f32)
        M = jnp.where(iota_r == iota_c, keep, 0.0) \
            + jnp.dot(wa_full, s, preferred_element_type=f32)
        k_ref[:, :D] = Wm_f
        k_ref[:, D:2 * D] = Ga_f
        k_ref[:, 2 * D:] = M
        bmf_ref[...] = jnp.dot(bm, s, preferred_element_type=f32)
        baf_ref[...] = jnp.dot(ba, s, preferred_element_type=f32)

    x = x_ref[...]
    mul_f = jnp.dot(x, k_ref[:, :D], preferred_element_type=f32) + bmf_ref[...]
    am_f = jnp.dot(x, k_ref[:, D:2 * D], preferred_element_type=f32)
    base_f = jnp.dot(x, k_ref[:, 2 * D:], preferred_element_type=f32) + baf_ref[...]
    out_ref[...] = mul_f * am_f + base_f


def kernel(x, W_mul, b_mul, W_add, b_add, indices, idx_A, idx_B):
    n = x.shape[0]
    grid = n // BLOCK

    vec = pl.BlockSpec((H,), lambda i: (0,))
    mat = pl.BlockSpec((H, H), lambda i: (0, 0))
    return pl.pallas_call(
        _body,
        grid=(grid,),
        in_specs=[
            vec,
            vec,
            vec,
            mat,
            mat,
            vec,
            vec,
            pl.BlockSpec((BLOCK, D), lambda i: (i, 0)),
        ],
        out_specs=pl.BlockSpec((BLOCK, D), lambda i: (i, 0)),
        out_shape=jax.ShapeDtypeStruct((n, D), jnp.float32),
        scratch_shapes=[
            pltpu.VMEM((D, 3 * D), jnp.float32),
            pltpu.VMEM((1, D), jnp.float32),
            pltpu.VMEM((1, D), jnp.float32),
        ],
    )(idx_A, idx_B, indices, W_mul, W_add, b_mul, b_add, x)


# speedup vs baseline: 1.2525x; 1.0150x over previous
import jax
import jax.numpy as jnp
from jax import lax
from jax.experimental import pallas as pl
from jax.experimental.pallas import tpu as pltpu

D = 128
H = 64
BLOCK = 8192


def _body(idxa_ref, idxb_ref, ind_ref, wm_ref, wa_ref, bm_ref, ba_ref,
          x_ref, out_ref, k_ref, bmf_ref, baf_ref):
    f32 = jnp.float32

    @pl.when(pl.program_id(0) == 0)
    def _prep():
        idxa = idxa_ref[...].reshape(1, H)
        idxb = idxb_ref[...].reshape(1, H)
        ind = ind_ref[...].reshape(1, H)
        bm = bm_ref[...].reshape(1, H)
        ba = ba_ref[...].reshape(1, H)
        iota_dh = lax.broadcasted_iota(jnp.int32, (D, H), 0)
        ga = (iota_dh == idxa).astype(f32)
        gb = (iota_dh == idxb).astype(f32)
        s = (iota_dh == ind).astype(f32).T
        t_dims = (((1,), (1,)), ((), ()))
        wm_full = lax.dot_general(gb, wm_ref[...], t_dims, preferred_element_type=f32)
        wa_full = lax.dot_general(gb, wa_ref[...], t_dims, preferred_element_type=f32)
        keep = 1.0 - jnp.sum(s, axis=0, keepdims=True)
        iota_r = lax.broadcasted_iota(jnp.int32, (D, D), 0)
        iota_c = lax.broadcasted_iota(jnp.int32, (D, D), 1)
        Wm_f = jnp.dot(wm_full, s, preferred_element_type=f32)
        Ga_f = jnp.dot(ga, s, preferred_element_type=f32)
        M = jnp.where(iota_r == iota_c, keep, 0.0) \
            + jnp.dot(wa_full, s, preferred_element_type=f32)
        k_ref[...] = jnp.concatenate([Wm_f, Ga_f, M], axis=1)
        bmf_ref[...] = jnp.dot(bm, s, preferred_element_type=f32)
        baf_ref[...] = jnp.dot(ba, s, preferred_element_type=f32)

    x = x_ref[...]
    acc = jnp.dot(x, k_ref[...], preferred_element_type=f32)
    out_ref[...] = (acc[:, :D] + bmf_ref[...]) * acc[:, D:2 * D] \
        + acc[:, 2 * D:] + baf_ref[...]


def kernel(x, W_mul, b_mul, W_add, b_add, indices, idx_A, idx_B):
    n = x.shape[0]
    grid = n // BLOCK

    vec = pl.BlockSpec((H,), lambda i: (0,))
    mat = pl.BlockSpec((H, H), lambda i: (0, 0))
    return pl.pallas_call(
        _body,
        grid=(grid,),
        in_specs=[
            vec,
            vec,
            vec,
            mat,
            mat,
            vec,
            vec,
            pl.BlockSpec((BLOCK, D), lambda i: (i, 0)),
        ],
        out_specs=pl.BlockSpec((BLOCK, D), lambda i: (i, 0)),
        out_shape=jax.ShapeDtypeStruct((n, D), jnp.float32),
        scratch_shapes=[
            pltpu.VMEM((D, 3 * D), jnp.float32),
            pltpu.VMEM((1, D), jnp.float32),
            pltpu.VMEM((1, D), jnp.float32),
        ],
    )(idx_A, idx_B, indices, W_mul, W_add, b_mul, b_add, x)
